# Initial kernel scaffold; baseline (speedup 1.0000x reference)
import functools

import jax
import jax.numpy as jnp
from jax import lax
from jax.experimental import pallas as pl
from jax.experimental.pallas import tpu as pltpu
from jax.experimental.pallas import tpu_sc as plsc

N_NODES = 10000
N_EDGES = 320000
NODE_FEAT = 128
INPUT_ENC = 64

NT = 16
L = 16
NP = 10240
NSL = NP // NT
EP = 327680
EPT = EP // NT
MPNN_STEPS = 3


def _sc_body(src_hbm, dst_hbm, w_hbm,
             src_v, dst_v, ufull_v, acc_v, red_v,
             wsl_v, cinv_v, usl_v, stage_sh, ush_sh):
    tid = lax.axis_index("s")
    ebase = tid * EPT
    nbase = tid * NSL
    zeros16 = jnp.zeros((L,), jnp.float32)
    ones16 = jnp.ones((L,), jnp.float32)

    pltpu.sync_copy(src_hbm.at[pl.ds(ebase, EPT)], src_v)
    pltpu.sync_copy(dst_hbm.at[pl.ds(ebase, EPT)], dst_v)

    def zero_acc():
        def body(i, _):
            acc_v[pl.ds(i * L, L)] = zeros16
            return 0
        lax.fori_loop(0, NP // L, body, 0)

    def reduce_chunk(j):
        s = red_v[0, pl.ds(j * L, L)]
        for r in range(1, NT):
            s = s + red_v[r, pl.ds(j * L, L)]
        return s

    zero_acc()

    def count_body(i, _):
        d = dst_v[pl.ds(i * L, L)]
        plsc.addupdate_scatter(acc_v, [d], ones16)
        return 0
    lax.fori_loop(0, EPT // L, count_body, 0)

    pltpu.sync_copy(acc_v, stage_sh.at[tid])
    plsc.subcore_barrier()
    pltpu.sync_copy(stage_sh.at[:, pl.ds(nbase, NSL)], red_v)

    def init_body(j, _):
        c = jnp.maximum(reduce_chunk(j), 1.0)
        cinv = 1.0 / c
        cinv_v[pl.ds(j * L, L)] = cinv
        gidx = nbase + j * L + lax.iota(jnp.int32, L)
        w = jnp.where(gidx < N_NODES, jnp.float32(1.0 / N_NODES), 0.0)
        wsl_v[pl.ds(j * L, L)] = w
        usl_v[pl.ds(j * L, L)] = w * cinv
        return 0
    lax.fori_loop(0, NSL // L, init_body, 0)

    pltpu.sync_copy(usl_v, ush_sh.at[pl.ds(nbase, NSL)])
    plsc.subcore_barrier()

    for step in range(MPNN_STEPS):
        pltpu.sync_copy(ush_sh, ufull_v)
        zero_acc()

        def edge_body(i, _):
            d = dst_v[pl.ds(i * L, L)]
            s = src_v[pl.ds(i * L, L)]
            vals = plsc.load_gather(ufull_v, [d])
            plsc.addupdate_scatter(acc_v, [s], vals)
            return 0
        lax.fori_loop(0, EPT // L, edge_body, 0)

        pltpu.sync_copy(acc_v, stage_sh.at[tid])
        plsc.subcore_barrier()
        pltpu.sync_copy(stage_sh.at[:, pl.ds(nbase, NSL)], red_v)

        def upd_body(j, _):
            w = (wsl_v[pl.ds(j * L, L)] + reduce_chunk(j)) * 0.5
            wsl_v[pl.ds(j * L, L)] = w
            usl_v[pl.ds(j * L, L)] = w * cinv_v[pl.ds(j * L, L)]
            return 0
        lax.fori_loop(0, NSL // L, upd_body, 0)

        if step < MPNN_STEPS - 1:
            pltpu.sync_copy(usl_v, ush_sh.at[pl.ds(nbase, NSL)])
            plsc.subcore_barrier()
        else:
            pltpu.sync_copy(wsl_v, w_hbm.at[pl.ds(nbase, NSL)])


def _sc_propagate(srcp, dstp):
    mesh = plsc.VectorSubcoreMesh(core_axis_name="c", subcore_axis_name="s",
                                  num_cores=1)
    kern = pl.kernel(
        _sc_body,
        out_type=jax.ShapeDtypeStruct((NP,), jnp.float32),
        mesh=mesh,
        scratch_types=[
            pltpu.VMEM((EPT,), jnp.int32),
            pltpu.VMEM((EPT,), jnp.int32),
            pltpu.VMEM((NP,), jnp.float32),
            pltpu.VMEM((NP,), jnp.float32),
            pltpu.VMEM((NT, NSL), jnp.float32),
            pltpu.VMEM((NSL,), jnp.float32),
            pltpu.VMEM((NSL,), jnp.float32),
            pltpu.VMEM((NSL,), jnp.float32),
            pltpu.VMEM_SHARED((NT, NP), jnp.float32),
            pltpu.VMEM_SHARED((NP,), jnp.float32),
        ],
    )
    return kern(srcp, dstp)


def _tc_body(x_ref, w_ref, win_ref, b_ref, wp_ref, bp_ref, out_ref):
    h = lax.dot_general(x_ref[...], win_ref[...],
                        (((1,), (1,)), ((), ())),
                        preferred_element_type=jnp.float32)
    h = jnp.maximum(h + b_ref[...], 0.0)
    s = jnp.sum(h * w_ref[...], axis=0, keepdims=True)
    out_ref[...] = jnp.sum(s * wp_ref[...], axis=1, keepdims=True) + bp_ref[...]


def kernel(x, edge_index, W_in, b_in, W_pred, b_pred):
    ei = edge_index.astype(jnp.int32)
    pad = jnp.full((EP - N_EDGES,), N_NODES, jnp.int32)
    srcp = jnp.concatenate([ei[0], pad])
    dstp = jnp.concatenate([ei[1], pad])

    w3 = _sc_propagate(srcp, dstp)

    xp = jnp.pad(x, ((0, NP - N_NODES), (0, 0)))
    out = pl.pallas_call(
        _tc_body,
        out_shape=jax.ShapeDtypeStruct((1, 1), jnp.float32),
    )(xp, w3.reshape(NP, 1), W_in, b_in.reshape(1, INPUT_ENC),
      W_pred[:, :INPUT_ENC], b_pred.reshape(1, 1))
    return out.reshape(1)


# speedup vs baseline: 36.2555x; 36.2555x over previous
import functools

import jax
import jax.numpy as jnp
from jax import lax
from jax.experimental import pallas as pl
from jax.experimental.pallas import tpu as pltpu
from jax.experimental.pallas import tpu_sc as plsc

N_NODES = 10000
N_EDGES = 320000
NODE_FEAT = 128
INPUT_ENC = 64

NT = 16
L = 16
NP = 10240
NSL = NP // NT
EP = 327680
EPT = EP // NT
MPNN_STEPS = 3


def _sc_body(src_hbm, dst_hbm, w_hbm,
             src_v, dst_v, ufull_v, acc_v, red_v,
             wsl_v, cinv_v, usl_v, stage_sh, ush_sh):
    tid = lax.axis_index("s")
    ebase = tid * EPT
    nbase = tid * NSL
    zeros16 = jnp.zeros((L,), jnp.float32)
    ones16 = jnp.ones((L,), jnp.float32)

    pltpu.sync_copy(src_hbm.at[pl.ds(ebase, EPT)], src_v)
    pltpu.sync_copy(dst_hbm.at[pl.ds(ebase, EPT)], dst_v)

    def zero_acc():
        def body(i, _):
            acc_v[pl.ds(i * L, L)] = zeros16
            return 0
        lax.fori_loop(0, NP // L, body, 0)

    def reduce_chunk(j):
        s = red_v[0, pl.ds(j * L, L)]
        for r in range(1, NT):
            s = s + red_v[r, pl.ds(j * L, L)]
        return s

    zero_acc()

    def count_body(i, _):
        d = dst_v[pl.ds(i * L, L)]
        plsc.addupdate_scatter(acc_v, [d], ones16)
        return 0
    lax.fori_loop(0, EPT // L, count_body, 0)

    pltpu.sync_copy(acc_v, stage_sh.at[tid])
    plsc.subcore_barrier()
    pltpu.sync_copy(stage_sh.at[:, pl.ds(nbase, NSL)], red_v)

    def init_body(j, _):
        c = jnp.maximum(reduce_chunk(j), 1.0)
        cinv = 1.0 / c
        cinv_v[pl.ds(j * L, L)] = cinv
        gidx = nbase + j * L + lax.iota(jnp.int32, L)
        w = jnp.where(gidx < N_NODES, jnp.float32(1.0 / N_NODES), 0.0)
        wsl_v[pl.ds(j * L, L)] = w
        usl_v[pl.ds(j * L, L)] = w * cinv
        return 0
    lax.fori_loop(0, NSL // L, init_body, 0)

    pltpu.sync_copy(usl_v, ush_sh.at[pl.ds(nbase, NSL)])
    plsc.subcore_barrier()

    for step in range(MPNN_STEPS):
        pltpu.sync_copy(ush_sh, ufull_v)
        zero_acc()

        def edge_body(i, _):
            d = dst_v[pl.ds(i * L, L)]
            s = src_v[pl.ds(i * L, L)]
            vals = plsc.load_gather(ufull_v, [d])
            plsc.addupdate_scatter(acc_v, [s], vals)
            return 0
        lax.fori_loop(0, EPT // L, edge_body, 0)

        pltpu.sync_copy(acc_v, stage_sh.at[tid])
        plsc.subcore_barrier()
        pltpu.sync_copy(stage_sh.at[:, pl.ds(nbase, NSL)], red_v)

        def upd_body(j, _):
            w = (wsl_v[pl.ds(j * L, L)] + reduce_chunk(j)) * 0.5
            wsl_v[pl.ds(j * L, L)] = w
            usl_v[pl.ds(j * L, L)] = w * cinv_v[pl.ds(j * L, L)]
            return 0
        lax.fori_loop(0, NSL // L, upd_body, 0)

        if step < MPNN_STEPS - 1:
            pltpu.sync_copy(usl_v, ush_sh.at[pl.ds(nbase, NSL)])
            plsc.subcore_barrier()
        else:
            pltpu.sync_copy(wsl_v, w_hbm.at[pl.ds(nbase, NSL)])


def _sc_propagate(srcp, dstp):
    mesh = plsc.VectorSubcoreMesh(core_axis_name="c", subcore_axis_name="s",
                                  num_cores=1)
    kern = pl.kernel(
        _sc_body,
        out_type=jax.ShapeDtypeStruct((NP,), jnp.float32),
        mesh=mesh,
        compiler_params=pltpu.CompilerParams(needs_layout_passes=False),
        scratch_types=[
            pltpu.VMEM((EPT,), jnp.int32),
            pltpu.VMEM((EPT,), jnp.int32),
            pltpu.VMEM((NP,), jnp.float32),
            pltpu.VMEM((NP,), jnp.float32),
            pltpu.VMEM((NT, NSL), jnp.float32),
            pltpu.VMEM((NSL,), jnp.float32),
            pltpu.VMEM((NSL,), jnp.float32),
            pltpu.VMEM((NSL,), jnp.float32),
            pltpu.VMEM_SHARED((NT, NP), jnp.float32),
            pltpu.VMEM_SHARED((NP,), jnp.float32),
        ],
    )
    return kern(srcp, dstp)


def _tc_body(x_ref, w_ref, win_ref, b_ref, wp_ref, bp_ref, out_ref):
    h = lax.dot_general(x_ref[...], win_ref[...],
                        (((1,), (1,)), ((), ())),
                        preferred_element_type=jnp.float32)
    h = jnp.maximum(h + b_ref[...], 0.0)
    s = jnp.sum(h * w_ref[...], axis=0, keepdims=True)
    out_ref[...] = jnp.sum(s * wp_ref[...], axis=1, keepdims=True) + bp_ref[...]


def kernel(x, edge_index, W_in, b_in, W_pred, b_pred):
    ei = edge_index.astype(jnp.int32)
    pad = jnp.full((EP - N_EDGES,), N_NODES, jnp.int32)
    srcp = jnp.concatenate([ei[0], pad])
    dstp = jnp.concatenate([ei[1], pad])

    w3 = _sc_propagate(srcp, dstp)

    xp = jnp.pad(x, ((0, NP - N_NODES), (0, 0)))
    out = pl.pallas_call(
        _tc_body,
        out_shape=jax.ShapeDtypeStruct((1, 1), jnp.float32),
    )(xp, w3.reshape(NP, 1), W_in, b_in.reshape(1, INPUT_ENC),
      W_pred[:, :INPUT_ENC], b_pred.reshape(1, 1))
    return out.reshape(1)


# speedup vs baseline: 49.9681x; 1.3782x over previous
import jax
import jax.numpy as jnp
from jax import lax
from jax.experimental import pallas as pl
from jax.experimental.pallas import tpu as pltpu
from jax.experimental.pallas import tpu_sc as plsc

N_NODES = 10000
N_EDGES = 320000
NODE_FEAT = 128
INPUT_ENC = 64

NT = 16
L = 16
NP = 10240
NSL = NP // NT
EPT = N_EDGES // NT
UE = 10
MPNN_STEPS = 3


def _sc_body(src_hbm, dst_hbm, w_hbm,
             src_v, dst_v, ufull_v, acc_v, red_v,
             wsl_v, cinv_v, usl_v, stage_sh, ush_sh):
    tid = lax.axis_index("s")
    ebase = tid * EPT
    nbase = tid * NSL
    zeros16 = jnp.zeros((L,), jnp.float32)
    ones16 = jnp.ones((L,), jnp.float32)

    pltpu.sync_copy(src_hbm.at[pl.ds(ebase, EPT)], src_v)
    pltpu.sync_copy(dst_hbm.at[pl.ds(ebase, EPT)], dst_v)

    def zero_acc():
        def body(i, _):
            for k in range(8):
                acc_v[pl.ds(i * (8 * L) + k * L, L)] = zeros16
            return 0
        lax.fori_loop(0, NP // (8 * L), body, 0)

    def reduce_chunk(j):
        s = red_v[0, pl.ds(j * L, L)]
        for r in range(1, NT):
            s = s + red_v[r, pl.ds(j * L, L)]
        return s

    zero_acc()

    def count_body(i, _):
        for k in range(UE):
            d = dst_v[pl.ds(i * (UE * L) + k * L, L)]
            plsc.addupdate_scatter(acc_v, [d], ones16)
        return 0
    lax.fori_loop(0, EPT // (UE * L), count_body, 0)

    pltpu.sync_copy(acc_v, stage_sh.at[tid])
    plsc.subcore_barrier()
    pltpu.sync_copy(stage_sh.at[:, pl.ds(nbase, NSL)], red_v)

    def init_body(j, _):
        c = jnp.maximum(reduce_chunk(j), 1.0)
        cinv = 1.0 / c
        cinv_v[pl.ds(j * L, L)] = cinv
        gidx = nbase + j * L + lax.iota(jnp.int32, L)
        w = jnp.where(gidx < N_NODES, jnp.float32(1.0 / N_NODES), 0.0)
        wsl_v[pl.ds(j * L, L)] = w
        usl_v[pl.ds(j * L, L)] = w * cinv
        return 0
    lax.fori_loop(0, NSL // L, init_body, 0)

    pltpu.sync_copy(usl_v, ush_sh.at[pl.ds(nbase, NSL)])
    plsc.subcore_barrier()

    for step in range(MPNN_STEPS):
        pltpu.sync_copy(ush_sh, ufull_v)
        zero_acc()

        def edge_body(i, _):
            for k in range(UE):
                d = dst_v[pl.ds(i * (UE * L) + k * L, L)]
                s = src_v[pl.ds(i * (UE * L) + k * L, L)]
                vals = plsc.load_gather(ufull_v, [d])
                plsc.addupdate_scatter(acc_v, [s], vals)
            return 0
        lax.fori_loop(0, EPT // (UE * L), edge_body, 0)

        pltpu.sync_copy(acc_v, stage_sh.at[tid])
        plsc.subcore_barrier()
        pltpu.sync_copy(stage_sh.at[:, pl.ds(nbase, NSL)], red_v)

        def upd_body(j, _):
            w = (wsl_v[pl.ds(j * L, L)] + reduce_chunk(j)) * 0.5
            wsl_v[pl.ds(j * L, L)] = w
            usl_v[pl.ds(j * L, L)] = w * cinv_v[pl.ds(j * L, L)]
            return 0
        lax.fori_loop(0, NSL // L, upd_body, 0)

        if step < MPNN_STEPS - 1:
            pltpu.sync_copy(usl_v, ush_sh.at[pl.ds(nbase, NSL)])
            plsc.subcore_barrier()
        else:
            pltpu.sync_copy(wsl_v, w_hbm.at[pl.ds(nbase, NSL)])


def _sc_propagate(srcp, dstp):
    mesh = plsc.VectorSubcoreMesh(core_axis_name="c", subcore_axis_name="s",
                                  num_cores=1)
    kern = pl.kernel(
        _sc_body,
        out_type=jax.ShapeDtypeStruct((NP,), jnp.float32),
        mesh=mesh,
        compiler_params=pltpu.CompilerParams(needs_layout_passes=False),
        scratch_types=[
            pltpu.VMEM((EPT,), jnp.int32),
            pltpu.VMEM((EPT,), jnp.int32),
            pltpu.VMEM((NP,), jnp.float32),
            pltpu.VMEM((NP,), jnp.float32),
            pltpu.VMEM((NT, NSL), jnp.float32),
            pltpu.VMEM((NSL,), jnp.float32),
            pltpu.VMEM((NSL,), jnp.float32),
            pltpu.VMEM((NSL,), jnp.float32),
            pltpu.VMEM_SHARED((NT, NP), jnp.float32),
            pltpu.VMEM_SHARED((NP,), jnp.float32),
        ],
    )
    return kern(srcp, dstp)


def _tc_body(x_ref, w_ref, win_ref, b_ref, wp_ref, bp_ref, out_ref):
    h = lax.dot_general(x_ref[...], win_ref[...],
                        (((1,), (1,)), ((), ())),
                        preferred_element_type=jnp.float32)
    h = jnp.maximum(h + b_ref[...], 0.0)
    s = jnp.sum(h * w_ref[...], axis=0, keepdims=True)
    out_ref[...] = jnp.sum(s * wp_ref[...], axis=1, keepdims=True) + bp_ref[...]


def kernel(x, edge_index, W_in, b_in, W_pred, b_pred):
    ei32 = edge_index.astype(jnp.int32)

    w3 = _sc_propagate(ei32[0], ei32[1])

    out = pl.pallas_call(
        _tc_body,
        out_shape=jax.ShapeDtypeStruct((1, 1), jnp.float32),
    )(x, w3[:N_NODES].reshape(N_NODES, 1), W_in, b_in.reshape(1, INPUT_ENC),
      W_pred[:, :INPUT_ENC], b_pred.reshape(1, 1))
    return out.reshape(1)


# speedup vs baseline: 59.2066x; 1.1849x over previous
import jax
import jax.numpy as jnp
from jax import lax
from jax.experimental import pallas as pl
from jax.experimental.pallas import tpu as pltpu
from jax.experimental.pallas import tpu_sc as plsc

N_NODES = 10000
N_EDGES = 320000
NODE_FEAT = 128
INPUT_ENC = 64

NT = 16
L = 16
NP = 10240
NSL = NP // NT
EPT = N_EDGES // NT
UE = 10
MPNN_STEPS = 3


def _sc_body(ei_hbm, w_hbm,
             src_v, dst_v, ufull_v, acc_v, red_v,
             wsl_v, cinv_v, usl_v, stage_sh, ush_sh):
    tid = lax.axis_index("s")
    ebase = tid * EPT
    nbase = tid * NSL
    zeros16 = jnp.zeros((L,), jnp.float32)
    ones16 = jnp.ones((L,), jnp.float32)

    pltpu.sync_copy(ei_hbm.at[pl.ds(ebase, EPT)], src_v)
    pltpu.sync_copy(ei_hbm.at[pl.ds(N_EDGES + ebase, EPT)], dst_v)

    def zero_acc():
        def body(i, _):
            for k in range(8):
                acc_v[pl.ds(i * (8 * L) + k * L, L)] = zeros16
            return 0
        lax.fori_loop(0, NP // (8 * L), body, 0)

    def reduce_chunk(j):
        s = red_v[0, pl.ds(j * L, L)]
        for r in range(1, NT):
            s = s + red_v[r, pl.ds(j * L, L)]
        return s

    zero_acc()

    def count_body(i, _):
        for k in range(UE):
            d = dst_v[pl.ds(i * (UE * L) + k * L, L)]
            plsc.addupdate_scatter(acc_v, [d], ones16)
        return 0
    lax.fori_loop(0, EPT // (UE * L), count_body, 0)

    pltpu.sync_copy(acc_v, stage_sh.at[tid])
    plsc.subcore_barrier()
    pltpu.sync_copy(stage_sh.at[:, pl.ds(nbase, NSL)], red_v)

    def init_body(j, _):
        c = jnp.maximum(reduce_chunk(j), 1.0)
        cinv = 1.0 / c
        cinv_v[pl.ds(j * L, L)] = cinv
        gidx = nbase + j * L + lax.iota(jnp.int32, L)
        w = jnp.where(gidx < N_NODES, jnp.float32(1.0 / N_NODES), 0.0)
        wsl_v[pl.ds(j * L, L)] = w
        usl_v[pl.ds(j * L, L)] = w * cinv
        return 0
    lax.fori_loop(0, NSL // L, init_body, 0)

    pltpu.sync_copy(usl_v, ush_sh.at[pl.ds(nbase, NSL)])
    plsc.subcore_barrier()

    for step in range(MPNN_STEPS):
        pltpu.sync_copy(ush_sh, ufull_v)
        zero_acc()

        def edge_body(i, _):
            for k in range(UE):
                d = dst_v[pl.ds(i * (UE * L) + k * L, L)]
                s = src_v[pl.ds(i * (UE * L) + k * L, L)]
                vals = plsc.load_gather(ufull_v, [d])
                plsc.addupdate_scatter(acc_v, [s], vals)
            return 0
        lax.fori_loop(0, EPT // (UE * L), edge_body, 0)

        pltpu.sync_copy(acc_v, stage_sh.at[tid])
        plsc.subcore_barrier()
        pltpu.sync_copy(stage_sh.at[:, pl.ds(nbase, NSL)], red_v)

        def upd_body(j, _):
            w = (wsl_v[pl.ds(j * L, L)] + reduce_chunk(j)) * 0.5
            wsl_v[pl.ds(j * L, L)] = w
            usl_v[pl.ds(j * L, L)] = w * cinv_v[pl.ds(j * L, L)]
            return 0
        lax.fori_loop(0, NSL // L, upd_body, 0)

        if step < MPNN_STEPS - 1:
            pltpu.sync_copy(usl_v, ush_sh.at[pl.ds(nbase, NSL)])
            plsc.subcore_barrier()
        else:
            pltpu.sync_copy(wsl_v, w_hbm.at[pl.ds(nbase, NSL)])


def _sc_propagate(ei_flat):
    mesh = plsc.VectorSubcoreMesh(core_axis_name="c", subcore_axis_name="s",
                                  num_cores=1)
    kern = pl.kernel(
        _sc_body,
        out_type=jax.ShapeDtypeStruct((NP,), jnp.float32),
        mesh=mesh,
        compiler_params=pltpu.CompilerParams(needs_layout_passes=False),
        scratch_types=[
            pltpu.VMEM((EPT,), jnp.int32),
            pltpu.VMEM((EPT,), jnp.int32),
            pltpu.VMEM((NP,), jnp.float32),
            pltpu.VMEM((NP,), jnp.float32),
            pltpu.VMEM((NT, NSL), jnp.float32),
            pltpu.VMEM((NSL,), jnp.float32),
            pltpu.VMEM((NSL,), jnp.float32),
            pltpu.VMEM((NSL,), jnp.float32),
            pltpu.VMEM_SHARED((NT, NP), jnp.float32),
            pltpu.VMEM_SHARED((NP,), jnp.float32),
        ],
    )
    return kern(ei_flat)


def _tc_body(x_ref, w_ref, win_ref, b_ref, wp_ref, bp_ref, out_ref):
    h = lax.dot_general(x_ref[...], win_ref[...],
                        (((1,), (1,)), ((), ())),
                        preferred_element_type=jnp.float32)
    h = jnp.maximum(h + b_ref[...], 0.0)
    s = lax.dot_general(w_ref[...], h,
                        (((1,), (0,)), ((), ())),
                        preferred_element_type=jnp.float32)
    out_ref[...] = jnp.sum(s * wp_ref[...], axis=1, keepdims=True) + bp_ref[...]


def kernel(x, edge_index, W_in, b_in, W_pred, b_pred):
    ei_flat = edge_index.astype(jnp.int32).reshape(2 * N_EDGES)

    w3 = _sc_propagate(ei_flat)

    out = pl.pallas_call(
        _tc_body,
        out_shape=jax.ShapeDtypeStruct((1, 1), jnp.float32),
    )(x, w3[:N_NODES].reshape(1, N_NODES), W_in, b_in.reshape(1, INPUT_ENC),
      W_pred[:, :INPUT_ENC], b_pred.reshape(1, 1))
    return out.reshape(1)


# speedup vs baseline: 59.2191x; 1.0002x over previous
import jax
import jax.numpy as jnp
from jax import lax
from jax.experimental import pallas as pl
from jax.experimental.pallas import tpu as pltpu
from jax.experimental.pallas import tpu_sc as plsc

N_NODES = 10000
N_EDGES = 320000
NODE_FEAT = 128
INPUT_ENC = 64

NT = 16
L = 16
NP = 10240
NSL = NP // NT
EPT = 20480
EPT_LAST = N_EDGES - 15 * EPT
UE = 10
MPNN_STEPS = 3


def _sc_body(ei_hbm, w_hbm,
             sd_v, ufull_v, acc_v, red_v,
             wsl_v, cinv_v, usl_v, stage_sh, ush_sh):
    tid = lax.axis_index("s")
    nbase = tid * NSL
    zeros16 = jnp.zeros((L,), jnp.float32)
    ones16 = jnp.ones((L,), jnp.float32)
    n_eiters = jnp.where(tid < NT - 1, EPT // (UE * L), EPT_LAST // (UE * L))

    with jax.named_scope("edge_load"):
        @pl.when(tid < NT - 1)
        def _():
            pltpu.sync_copy(ei_hbm.at[:, pl.ds(tid * EPT, EPT)], sd_v)

        @pl.when(tid == NT - 1)
        def _():
            pltpu.sync_copy(ei_hbm.at[:, pl.ds((NT - 1) * EPT, EPT_LAST)],
                            sd_v.at[:, pl.ds(0, EPT_LAST)])

    def zero_acc():
        def body(i, _):
            for k in range(8):
                acc_v[pl.ds(i * (8 * L) + k * L, L)] = zeros16
            return 0
        lax.fori_loop(0, NP // (8 * L), body, 0)

    def reduce_chunk(j):
        s = red_v[0, pl.ds(j * L, L)]
        for r in range(1, NT):
            s = s + red_v[r, pl.ds(j * L, L)]
        return s

    with jax.named_scope("counts"):
        zero_acc()

        def count_body(i, _):
            for k in range(UE):
                d = sd_v[1, pl.ds(i * (UE * L) + k * L, L)]
                plsc.addupdate_scatter(acc_v, [d], ones16)
            return 0
        lax.fori_loop(0, n_eiters, count_body, 0)

        pltpu.sync_copy(acc_v, stage_sh.at[tid])
        plsc.subcore_barrier()
        pltpu.sync_copy(stage_sh.at[:, pl.ds(nbase, NSL)], red_v)

        def init_body(j, _):
            c = jnp.maximum(reduce_chunk(j), 1.0)
            cinv = 1.0 / c
            cinv_v[pl.ds(j * L, L)] = cinv
            gidx = nbase + j * L + lax.iota(jnp.int32, L)
            w = jnp.where(gidx < N_NODES, jnp.float32(1.0 / N_NODES), 0.0)
            wsl_v[pl.ds(j * L, L)] = w
            usl_v[pl.ds(j * L, L)] = w * cinv
            return 0
        lax.fori_loop(0, NSL // L, init_body, 0)

        pltpu.sync_copy(usl_v, ush_sh.at[pl.ds(nbase, NSL)])
        plsc.subcore_barrier()

    for step in range(MPNN_STEPS):
        with jax.named_scope(f"bcast{step}"):
            pltpu.sync_copy(ush_sh, ufull_v)
            zero_acc()

        with jax.named_scope(f"edges{step}"):
            def edge_body(i, _):
                for k in range(UE):
                    d = sd_v[1, pl.ds(i * (UE * L) + k * L, L)]
                    s = sd_v[0, pl.ds(i * (UE * L) + k * L, L)]
                    vals = plsc.load_gather(ufull_v, [d])
                    plsc.addupdate_scatter(acc_v, [s], vals)
                return 0
            lax.fori_loop(0, n_eiters, edge_body, 0)

        with jax.named_scope(f"reduce{step}"):
            pltpu.sync_copy(acc_v, stage_sh.at[tid])
            plsc.subcore_barrier()
            pltpu.sync_copy(stage_sh.at[:, pl.ds(nbase, NSL)], red_v)

            def upd_body(j, _):
                w = (wsl_v[pl.ds(j * L, L)] + reduce_chunk(j)) * 0.5
                wsl_v[pl.ds(j * L, L)] = w
                usl_v[pl.ds(j * L, L)] = w * cinv_v[pl.ds(j * L, L)]
                return 0
            lax.fori_loop(0, NSL // L, upd_body, 0)

            if step < MPNN_STEPS - 1:
                pltpu.sync_copy(usl_v, ush_sh.at[pl.ds(nbase, NSL)])
                plsc.subcore_barrier()
            else:
                pltpu.sync_copy(wsl_v, w_hbm.at[pl.ds(nbase, NSL)])


def _sc_propagate(ei_flat):
    mesh = plsc.VectorSubcoreMesh(core_axis_name="c", subcore_axis_name="s",
                                  num_cores=1)
    kern = pl.kernel(
        _sc_body,
        out_type=jax.ShapeDtypeStruct((NP,), jnp.float32),
        mesh=mesh,
        compiler_params=pltpu.CompilerParams(needs_layout_passes=False),
        scratch_types=[
            pltpu.VMEM((2, EPT), jnp.int32),
            pltpu.VMEM((NP,), jnp.float32),
            pltpu.VMEM((NP,), jnp.float32),
            pltpu.VMEM((NT, NSL), jnp.float32),
            pltpu.VMEM((NSL,), jnp.float32),
            pltpu.VMEM((NSL,), jnp.float32),
            pltpu.VMEM((NSL,), jnp.float32),
            pltpu.VMEM_SHARED((NT, NP), jnp.float32),
            pltpu.VMEM_SHARED((NP,), jnp.float32),
        ],
    )
    return kern(ei_flat)


def _tc_body(x_ref, w_ref, win_ref, b_ref, wp_ref, bp_ref, out_ref):
    h = lax.dot_general(x_ref[...], win_ref[...],
                        (((1,), (1,)), ((), ())),
                        preferred_element_type=jnp.float32)
    h = jnp.maximum(h + b_ref[...], 0.0)
    s = lax.dot_general(w_ref[...], h,
                        (((1,), (0,)), ((), ())),
                        preferred_element_type=jnp.float32)
    out_ref[...] = jnp.sum(s * wp_ref[...], axis=1, keepdims=True) + bp_ref[...]


def kernel(x, edge_index, W_in, b_in, W_pred, b_pred):
    w3 = _sc_propagate(edge_index.astype(jnp.int32))

    out = pl.pallas_call(
        _tc_body,
        out_shape=jax.ShapeDtypeStruct((1, 1), jnp.float32),
    )(x, w3[:N_NODES].reshape(1, N_NODES), W_in, b_in.reshape(1, INPUT_ENC),
      W_pred[:, :INPUT_ENC], b_pred.reshape(1, 1))
    return out.reshape(1)


# speedup vs baseline: 92.1179x; 1.5555x over previous
import jax
import jax.numpy as jnp
from jax import lax
from jax.experimental import pallas as pl
from jax.experimental.pallas import tpu as pltpu
from jax.experimental.pallas import tpu_sc as plsc

N_NODES = 10000
N_EDGES = 320000
NODE_FEAT = 128
INPUT_ENC = 64

NT = 16
L = 16
NP = 10240
NSL = NP // NT
EPT = 20480
EPT_LAST = N_EDGES - 15 * EPT
UE = 10
MPNN_STEPS = 3


def _sc_body(ei_hbm, w_hbm,
             sd_v, ufull_v, acc_v, red_v,
             wsl_v, cinv_v, usl_v, stage_sh, ush_sh):
    tid = lax.axis_index("s")
    nbase = tid * NSL
    zeros16 = jnp.zeros((L,), jnp.float32)
    ones16 = jnp.ones((L,), jnp.float32)
    n_chunks = jnp.where(tid < NT - 1, EPT // L, EPT_LAST // L)

    with jax.named_scope("edge_load"):
        @pl.when(tid < NT - 1)
        def _():
            pltpu.sync_copy(ei_hbm.at[:, pl.ds(tid * EPT, EPT)], sd_v)

        @pl.when(tid == NT - 1)
        def _():
            pltpu.sync_copy(ei_hbm.at[:, pl.ds((NT - 1) * EPT, EPT_LAST)],
                            sd_v.at[:, pl.ds(0, EPT_LAST)])

    def zero_acc():
        def body(i, _):
            for k in range(8):
                acc_v[pl.ds(i * (8 * L) + k * L, L)] = zeros16
            return 0
        lax.fori_loop(0, NP // (8 * L), body, 0)

    def reduce_chunk(j):
        s = red_v[0, pl.ds(j * L, L)]
        for r in range(1, NT):
            s = s + red_v[r, pl.ds(j * L, L)]
        return s

    with jax.named_scope("counts"):
        zero_acc()

        @plsc.parallel_loop(0, n_chunks, step=1, unroll=UE)
        def _(i):
            d = sd_v[1, pl.ds(i * L, L)]
            plsc.addupdate_scatter(acc_v, [d], ones16)

        pltpu.sync_copy(acc_v, stage_sh.at[tid])
        plsc.subcore_barrier()
        pltpu.sync_copy(stage_sh.at[:, pl.ds(nbase, NSL)], red_v)

        def init_body(j, _):
            c = jnp.maximum(reduce_chunk(j), 1.0)
            cinv = 1.0 / c
            cinv_v[pl.ds(j * L, L)] = cinv
            gidx = nbase + j * L + lax.iota(jnp.int32, L)
            w = jnp.where(gidx < N_NODES, jnp.float32(1.0 / N_NODES), 0.0)
            wsl_v[pl.ds(j * L, L)] = w
            usl_v[pl.ds(j * L, L)] = w * cinv
            return 0
        lax.fori_loop(0, NSL // L, init_body, 0)

        pltpu.sync_copy(usl_v, ush_sh.at[pl.ds(nbase, NSL)])
        plsc.subcore_barrier()

    for step in range(MPNN_STEPS):
        with jax.named_scope(f"bcast{step}"):
            pltpu.sync_copy(ush_sh, ufull_v)
            zero_acc()

        with jax.named_scope(f"edges{step}"):
            @plsc.parallel_loop(0, n_chunks, step=1, unroll=UE)
            def _(i):
                d = sd_v[1, pl.ds(i * L, L)]
                s = sd_v[0, pl.ds(i * L, L)]
                vals = plsc.load_gather(ufull_v, [d])
                plsc.addupdate_scatter(acc_v, [s], vals)

        with jax.named_scope(f"reduce{step}"):
            pltpu.sync_copy(acc_v, stage_sh.at[tid])
            plsc.subcore_barrier()
            pltpu.sync_copy(stage_sh.at[:, pl.ds(nbase, NSL)], red_v)

            def upd_body(j, _):
                w = (wsl_v[pl.ds(j * L, L)] + reduce_chunk(j)) * 0.5
                wsl_v[pl.ds(j * L, L)] = w
                usl_v[pl.ds(j * L, L)] = w * cinv_v[pl.ds(j * L, L)]
                return 0
            lax.fori_loop(0, NSL // L, upd_body, 0)

            if step < MPNN_STEPS - 1:
                pltpu.sync_copy(usl_v, ush_sh.at[pl.ds(nbase, NSL)])
                plsc.subcore_barrier()
            else:
                pltpu.sync_copy(wsl_v, w_hbm.at[pl.ds(nbase, NSL)])


def _sc_propagate(ei_flat):
    mesh = plsc.VectorSubcoreMesh(core_axis_name="c", subcore_axis_name="s",
                                  num_cores=1)
    kern = pl.kernel(
        _sc_body,
        out_type=jax.ShapeDtypeStruct((NP,), jnp.float32),
        mesh=mesh,
        compiler_params=pltpu.CompilerParams(needs_layout_passes=False),
        scratch_types=[
            pltpu.VMEM((2, EPT), jnp.int32),
            pltpu.VMEM((NP,), jnp.float32),
            pltpu.VMEM((NP,), jnp.float32),
            pltpu.VMEM((NT, NSL), jnp.float32),
            pltpu.VMEM((NSL,), jnp.float32),
            pltpu.VMEM((NSL,), jnp.float32),
            pltpu.VMEM((NSL,), jnp.float32),
            pltpu.VMEM_SHARED((NT, NP), jnp.float32),
            pltpu.VMEM_SHARED((NP,), jnp.float32),
        ],
    )
    return kern(ei_flat)


def _tc_body(x_ref, w_ref, win_ref, b_ref, wp_ref, bp_ref, out_ref):
    h = lax.dot_general(x_ref[...], win_ref[...],
                        (((1,), (1,)), ((), ())),
                        preferred_element_type=jnp.float32)
    h = jnp.maximum(h + b_ref[...], 0.0)
    s = lax.dot_general(w_ref[...], h,
                        (((1,), (0,)), ((), ())),
                        preferred_element_type=jnp.float32)
    out_ref[...] = jnp.sum(s * wp_ref[...], axis=1, keepdims=True) + bp_ref[...]


def kernel(x, edge_index, W_in, b_in, W_pred, b_pred):
    w3 = _sc_propagate(edge_index.astype(jnp.int32))

    out = pl.pallas_call(
        _tc_body,
        out_shape=jax.ShapeDtypeStruct((1, 1), jnp.float32),
    )(x, w3[:N_NODES].reshape(1, N_NODES), W_in, b_in.reshape(1, INPUT_ENC),
      W_pred[:, :INPUT_ENC], b_pred.reshape(1, 1))
    return out.reshape(1)


# speedup vs baseline: 99.9395x; 1.0849x over previous
import jax
import jax.numpy as jnp
from jax import lax
from jax.experimental import pallas as pl
from jax.experimental.pallas import tpu as pltpu
from jax.experimental.pallas import tpu_sc as plsc

N_NODES = 10000
N_EDGES = 320000
NODE_FEAT = 128
INPUT_ENC = 64

NT = 16
L = 16
NP = 10240
NSL = NP // NT
EPT = 20480
EPT_LAST = N_EDGES - 15 * EPT
UE = 10
MPNN_STEPS = 3


def _sc_body(ei_hbm, w_hbm,
             sd_v, ufull_v, acc_v, red_v,
             wsl_v, cinv_v, usl_v, stage_sh, ush_sh):
    tid = lax.axis_index("s")
    nbase = tid * NSL
    zeros16 = jnp.zeros((L,), jnp.float32)
    ones16 = jnp.ones((L,), jnp.float32)
    n_chunks = jnp.where(tid < NT - 1, EPT // L, EPT_LAST // L)

    with jax.named_scope("edge_load"):
        @pl.when(tid < NT - 1)
        def _():
            pltpu.sync_copy(ei_hbm.at[:, pl.ds(tid * EPT, EPT)], sd_v)

        @pl.when(tid == NT - 1)
        def _():
            pltpu.sync_copy(ei_hbm.at[:, pl.ds((NT - 1) * EPT, EPT_LAST)],
                            sd_v.at[:, pl.ds(0, EPT_LAST)])

    def zero_acc():
        def body(i, _):
            for k in range(8):
                acc_v[pl.ds(i * (8 * L) + k * L, L)] = zeros16
            return 0
        lax.fori_loop(0, NP // (8 * L), body, 0)

    def reduce_chunk(j):
        s = red_v[0, pl.ds(j * L, L)]
        for r in range(1, NT):
            s = s + red_v[r, pl.ds(j * L, L)]
        return s

    with jax.named_scope("counts"):
        zero_acc()

        @plsc.parallel_loop(0, n_chunks, step=1, unroll=UE)
        def _(i):
            d = sd_v[1, pl.ds(i * L, L)]
            plsc.addupdate_scatter(acc_v, [d], ones16)

        pltpu.sync_copy(acc_v, stage_sh.at[tid])
        plsc.subcore_barrier()
        pltpu.sync_copy(stage_sh.at[:, pl.ds(nbase, NSL)], red_v)

        @plsc.parallel_loop(0, NSL // L, step=1, unroll=4)
        def _(j):
            c = jnp.maximum(reduce_chunk(j), 1.0)
            cinv = 1.0 / c
            cinv_v[pl.ds(j * L, L)] = cinv
            gidx = nbase + j * L + lax.iota(jnp.int32, L)
            w = jnp.where(gidx < N_NODES, jnp.float32(1.0 / N_NODES), 0.0)
            wsl_v[pl.ds(j * L, L)] = w
            usl_v[pl.ds(j * L, L)] = w * cinv

        pltpu.sync_copy(usl_v, ush_sh.at[pl.ds(nbase, NSL)])
        plsc.subcore_barrier()

    for step in range(MPNN_STEPS):
        with jax.named_scope(f"bcast{step}"):
            pltpu.sync_copy(ush_sh, ufull_v)
            zero_acc()

        with jax.named_scope(f"edges{step}"):
            @plsc.parallel_loop(0, n_chunks, step=1, unroll=UE)
            def _(i):
                d = sd_v[1, pl.ds(i * L, L)]
                s = sd_v[0, pl.ds(i * L, L)]
                vals = plsc.load_gather(ufull_v, [d])
                plsc.addupdate_scatter(acc_v, [s], vals)

        with jax.named_scope(f"reduce{step}"):
            pltpu.sync_copy(acc_v, stage_sh.at[tid])
            plsc.subcore_barrier()
            pltpu.sync_copy(stage_sh.at[:, pl.ds(nbase, NSL)], red_v)

            @plsc.parallel_loop(0, NSL // L, step=1, unroll=4)
            def _(j):
                w = (wsl_v[pl.ds(j * L, L)] + reduce_chunk(j)) * 0.5
                wsl_v[pl.ds(j * L, L)] = w
                usl_v[pl.ds(j * L, L)] = w * cinv_v[pl.ds(j * L, L)]

            if step < MPNN_STEPS - 1:
                pltpu.sync_copy(usl_v, ush_sh.at[pl.ds(nbase, NSL)])
                plsc.subcore_barrier()
            else:
                pltpu.sync_copy(wsl_v, w_hbm.at[pl.ds(nbase, NSL)])


def _sc_propagate(ei_flat):
    mesh = plsc.VectorSubcoreMesh(core_axis_name="c", subcore_axis_name="s",
                                  num_cores=1)
    kern = pl.kernel(
        _sc_body,
        out_type=jax.ShapeDtypeStruct((NP,), jnp.float32),
        mesh=mesh,
        compiler_params=pltpu.CompilerParams(needs_layout_passes=False),
        scratch_types=[
            pltpu.VMEM((2, EPT), jnp.int32),
            pltpu.VMEM((NP,), jnp.float32),
            pltpu.VMEM((NP,), jnp.float32),
            pltpu.VMEM((NT, NSL), jnp.float32),
            pltpu.VMEM((NSL,), jnp.float32),
            pltpu.VMEM((NSL,), jnp.float32),
            pltpu.VMEM((NSL,), jnp.float32),
            pltpu.VMEM_SHARED((NT, NP), jnp.float32),
            pltpu.VMEM_SHARED((NP,), jnp.float32),
        ],
    )
    return kern(ei_flat)


def _enc_body(x_ref, win_ref, b_ref, h_ref):
    h = lax.dot_general(x_ref[...], win_ref[...],
                        (((1,), (1,)), ((), ())),
                        preferred_element_type=jnp.float32)
    h_ref[pl.ds(0, N_NODES), :] = jnp.maximum(h + b_ref[...], 0.0)
    h_ref[pl.ds(N_NODES, NP - N_NODES), :] = jnp.zeros(
        (NP - N_NODES, INPUT_ENC), jnp.float32)


def _fin_body(w_ref, h_ref, wp_ref, bp_ref, out_ref):
    s = lax.dot_general(w_ref[...], h_ref[...],
                        (((1,), (0,)), ((), ())),
                        preferred_element_type=jnp.float32)
    out_ref[...] = jnp.sum(s * wp_ref[...], axis=1, keepdims=True) + bp_ref[...]


def kernel(x, edge_index, W_in, b_in, W_pred, b_pred):
    h0p = pl.pallas_call(
        _enc_body,
        out_shape=jax.ShapeDtypeStruct((NP, INPUT_ENC), jnp.float32),
    )(x, W_in, b_in.reshape(1, INPUT_ENC))

    w3 = _sc_propagate(edge_index.astype(jnp.int32))

    out = pl.pallas_call(
        _fin_body,
        out_shape=jax.ShapeDtypeStruct((1, 1), jnp.float32),
    )(w3.reshape(1, NP), h0p, W_pred[:, :INPUT_ENC], b_pred.reshape(1, 1))
    return out.reshape(1)


# speedup vs baseline: 104.8748x; 1.0494x over previous
import jax
import jax.numpy as jnp
from jax import lax
from jax.experimental import pallas as pl
from jax.experimental.pallas import tpu as pltpu
from jax.experimental.pallas import tpu_sc as plsc

N_NODES = 10000
N_EDGES = 320000
NODE_FEAT = 128
INPUT_ENC = 64

NT = 16
L = 16
NP = 10240
NSL = NP // NT
EPT = 20480
EPT_LAST = N_EDGES - 15 * EPT
UE = 10
MPNN_STEPS = 3


def _sc_body(ei_hbm, w_hbm,
             sd_v, pk_v, ufull_v, acc_v, red_v,
             wsl_v, cinv_v, usl_v, stage_sh, ush_sh):
    tid = lax.axis_index("s")
    nbase = tid * NSL
    zeros16 = jnp.zeros((L,), jnp.float32)
    ones16 = jnp.ones((L,), jnp.float32)
    n_chunks = jnp.where(tid < NT - 1, EPT // L, EPT_LAST // L)

    with jax.named_scope("edge_load"):
        @pl.when(tid < NT - 1)
        def _():
            pltpu.sync_copy(ei_hbm.at[:, pl.ds(tid * EPT, EPT)], sd_v)

        @pl.when(tid == NT - 1)
        def _():
            pltpu.sync_copy(ei_hbm.at[:, pl.ds((NT - 1) * EPT, EPT_LAST)],
                            sd_v.at[:, pl.ds(0, EPT_LAST)])

    def zero_acc():
        def body(i, _):
            for k in range(8):
                acc_v[pl.ds(i * (8 * L) + k * L, L)] = zeros16
            return 0
        lax.fori_loop(0, NP // (8 * L), body, 0)

    def reduce_chunk(j):
        s = red_v[0, pl.ds(j * L, L)]
        for r in range(1, NT):
            s = s + red_v[r, pl.ds(j * L, L)]
        return s

    with jax.named_scope("counts"):
        zero_acc()

        @plsc.parallel_loop(0, n_chunks, step=1, unroll=UE)
        def _(i):
            d = sd_v[1, pl.ds(i * L, L)]
            s = sd_v[0, pl.ds(i * L, L)]
            plsc.addupdate_scatter(acc_v, [d], ones16)
            pk_v[pl.ds(i * L, L)] = s | (d << 16)

        pltpu.sync_copy(acc_v, stage_sh.at[tid])
        plsc.subcore_barrier()
        pltpu.sync_copy(stage_sh.at[:, pl.ds(nbase, NSL)], red_v)

        @plsc.parallel_loop(0, NSL // L, step=1, unroll=4)
        def _(j):
            c = jnp.maximum(reduce_chunk(j), 1.0)
            cinv = 1.0 / c
            cinv_v[pl.ds(j * L, L)] = cinv
            gidx = nbase + j * L + lax.iota(jnp.int32, L)
            w = jnp.where(gidx < N_NODES, jnp.float32(1.0 / N_NODES), 0.0)
            wsl_v[pl.ds(j * L, L)] = w
            usl_v[pl.ds(j * L, L)] = w * cinv

        pltpu.sync_copy(usl_v, ush_sh.at[pl.ds(nbase, NSL)])
        plsc.subcore_barrier()

    for step in range(MPNN_STEPS):
        with jax.named_scope(f"bcast{step}"):
            pltpu.sync_copy(ush_sh, ufull_v)
            zero_acc()

        with jax.named_scope(f"edges{step}"):
            @plsc.parallel_loop(0, n_chunks, step=1, unroll=UE)
            def _(i):
                pk = pk_v[pl.ds(i * L, L)]
                d = lax.shift_right_logical(pk, 16)
                s = pk & 0xFFFF
                vals = plsc.load_gather(ufull_v, [d])
                plsc.addupdate_scatter(acc_v, [s], vals)

        with jax.named_scope(f"reduce{step}"):
            pltpu.sync_copy(acc_v, stage_sh.at[tid])
            plsc.subcore_barrier()
            pltpu.sync_copy(stage_sh.at[:, pl.ds(nbase, NSL)], red_v)

            @plsc.parallel_loop(0, NSL // L, step=1, unroll=4)
            def _(j):
                w = (wsl_v[pl.ds(j * L, L)] + reduce_chunk(j)) * 0.5
                wsl_v[pl.ds(j * L, L)] = w
                usl_v[pl.ds(j * L, L)] = w * cinv_v[pl.ds(j * L, L)]

            if step < MPNN_STEPS - 1:
                pltpu.sync_copy(usl_v, ush_sh.at[pl.ds(nbase, NSL)])
                plsc.subcore_barrier()
            else:
                pltpu.sync_copy(wsl_v, w_hbm.at[pl.ds(nbase, NSL)])


def _sc_propagate(ei_flat):
    mesh = plsc.VectorSubcoreMesh(core_axis_name="c", subcore_axis_name="s",
                                  num_cores=1)
    kern = pl.kernel(
        _sc_body,
        out_type=jax.ShapeDtypeStruct((NP,), jnp.float32),
        mesh=mesh,
        compiler_params=pltpu.CompilerParams(needs_layout_passes=False),
        scratch_types=[
            pltpu.VMEM((2, EPT), jnp.int32),
            pltpu.VMEM((EPT,), jnp.int32),
            pltpu.VMEM((NP,), jnp.float32),
            pltpu.VMEM((NP,), jnp.float32),
            pltpu.VMEM((NT, NSL), jnp.float32),
            pltpu.VMEM((NSL,), jnp.float32),
            pltpu.VMEM((NSL,), jnp.float32),
            pltpu.VMEM((NSL,), jnp.float32),
            pltpu.VMEM_SHARED((NT, NP), jnp.float32),
            pltpu.VMEM_SHARED((NP,), jnp.float32),
        ],
    )
    return kern(ei_flat)


def _enc_body(x_ref, win_ref, b_ref, h_ref):
    h = lax.dot_general(win_ref[...], x_ref[...],
                        (((1,), (1,)), ((), ())),
                        preferred_element_type=jnp.float32)
    h_ref[...] = jnp.maximum(h + b_ref[...], 0.0)


def _fin_body(w_ref, h_ref, wp_ref, bp_ref, out_ref):
    wv = w_ref[...][:, :N_NODES]
    s = jnp.sum(h_ref[...] * wv, axis=1, keepdims=True)
    out_ref[...] = jnp.sum(s * wp_ref[...], axis=0, keepdims=True) + bp_ref[...]


def kernel(x, edge_index, W_in, b_in, W_pred, b_pred):
    h0t = pl.pallas_call(
        _enc_body,
        out_shape=jax.ShapeDtypeStruct((INPUT_ENC, N_NODES), jnp.float32),
    )(x, W_in, b_in.reshape(INPUT_ENC, 1))

    w3 = _sc_propagate(edge_index.astype(jnp.int32))

    out = pl.pallas_call(
        _fin_body,
        out_shape=jax.ShapeDtypeStruct((1, 1), jnp.float32),
    )(w3.reshape(1, NP), h0t, W_pred[:, :INPUT_ENC].reshape(INPUT_ENC, 1),
      b_pred.reshape(1, 1))
    return out.reshape(1)
